# baseline (speedup 1.0000x reference)
import functools

import numpy as np

import jax
import jax.numpy as jnp
from jax import lax
from jax.experimental import pallas as pl
from jax.experimental.pallas import tpu as pltpu
from jax.experimental.pallas import tpu_sc as plsc

N = 10000
E = 320000
DF = 128
DH = 128
DEF = 20
ACC_W = 144

NC, NS = 2, 16
NW = NC * NS
EPW = E // NW
K = 40
NBLK = EPW // K
KC = 80
DW = 64
NPAD = 10240
RPT = NPAD // NS
RZC = 128


def _tc_pre_body(child_ref, exists_ref, wct_ref, bc_ref, w1t_ref, w2t_ref,
                 a_ref, b_ref, psum_ref, esum_ref):
    i = pl.program_id(0)
    x = child_ref[...]
    cf = jnp.maximum(
        jnp.dot(x, wct_ref[...], preferred_element_type=jnp.float32)
        + bc_ref[...], 0.0) * exists_ref[...]
    a_ref[...] = jnp.dot(cf, w1t_ref[...], preferred_element_type=jnp.float32)
    b_ref[...] = jnp.dot(cf, w2t_ref[...], preferred_element_type=jnp.float32)
    ps = jnp.sum(cf, axis=0, keepdims=True)
    es = jnp.sum(exists_ref[...], axis=0, keepdims=True)

    @pl.when(i == 0)
    def _():
        psum_ref[...] = ps
        esum_ref[...] = es

    @pl.when(i > 0)
    def _():
        psum_ref[...] += ps
        esum_ref[...] += es


def _tc_pre(child, exists, wct, bc, w1t, w2t):
    blk = 2000
    grid = (N // blk,)
    return pl.pallas_call(
        _tc_pre_body,
        grid=grid,
        in_specs=[
            pl.BlockSpec((blk, DF), lambda i: (i, 0)),
            pl.BlockSpec((blk, 1), lambda i: (i, 0)),
            pl.BlockSpec((DF, DH), lambda i: (0, 0)),
            pl.BlockSpec((1, DH), lambda i: (0, 0)),
            pl.BlockSpec((DH, DH), lambda i: (0, 0)),
            pl.BlockSpec((DH, DH), lambda i: (0, 0)),
        ],
        out_specs=[
            pl.BlockSpec((blk, DH), lambda i: (i, 0)),
            pl.BlockSpec((blk, DH), lambda i: (i, 0)),
            pl.BlockSpec((1, DH), lambda i: (0, 0)),
            pl.BlockSpec((1, 1), lambda i: (0, 0)),
        ],
        out_shape=[
            jax.ShapeDtypeStruct((N, DH), jnp.float32),
            jax.ShapeDtypeStruct((N, DH), jnp.float32),
            jax.ShapeDtypeStruct((1, DH), jnp.float32),
            jax.ShapeDtypeStruct((1, 1), jnp.float32),
        ],
    )(child, exists, wct, bc, w1t, w2t)


def _tc_edgeproj_body(ef_ref, w3t_ref, bne_ref, c_ref):
    c_ref[...] = (
        jnp.dot(ef_ref[...], w3t_ref[...], preferred_element_type=jnp.float32)
        + bne_ref[...]).astype(jnp.bfloat16)


def _tc_edgeproj(ef, w3t, bne):
    blk = 8000
    return pl.pallas_call(
        _tc_edgeproj_body,
        grid=(E // blk,),
        in_specs=[
            pl.BlockSpec((blk, DEF), lambda i: (i, 0)),
            pl.BlockSpec((DEF, DH), lambda i: (0, 0)),
            pl.BlockSpec((1, DH), lambda i: (0, 0)),
        ],
        out_specs=pl.BlockSpec((blk, DH), lambda i: (i, 0)),
        out_shape=jax.ShapeDtypeStruct((E, DH), jnp.bfloat16),
    )(ef, w3t, bne)


def _tc_mid_body(p0_ref, p1_ref, cnt_ref, ones_ref, w1t_ref, w2t_ref,
                 a_ref, b_ref, psum_ref):
    i = pl.program_id(0)
    counts = lax.dot_general(cnt_ref[...], ones_ref[...],
                             (((0,), (0,)), ((), ())),
                             preferred_element_type=jnp.float32)
    cf = (p0_ref[...] + p1_ref[...]) / jnp.maximum(counts, 1.0)
    a_ref[...] = jnp.dot(cf, w1t_ref[...], preferred_element_type=jnp.float32)
    b_ref[...] = jnp.dot(cf, w2t_ref[...], preferred_element_type=jnp.float32)
    ps = jnp.sum(cf, axis=0, keepdims=True)

    @pl.when(i == 0)
    def _():
        psum_ref[...] = ps

    @pl.when(i > 0)
    def _():
        psum_ref[...] += ps


def _tc_mid(p0, p1, cnts, ones, w1t, w2t):
    blk = 2048
    return pl.pallas_call(
        _tc_mid_body,
        grid=(NPAD // blk,),
        in_specs=[
            pl.BlockSpec((blk, DH), lambda i: (i, 0)),
            pl.BlockSpec((blk, DH), lambda i: (i, 0)),
            pl.BlockSpec((NW, blk), lambda i: (0, i)),
            pl.BlockSpec((NW, 1), lambda i: (0, 0)),
            pl.BlockSpec((DH, DH), lambda i: (0, 0)),
            pl.BlockSpec((DH, DH), lambda i: (0, 0)),
        ],
        out_specs=[
            pl.BlockSpec((blk, DH), lambda i: (i, 0)),
            pl.BlockSpec((blk, DH), lambda i: (i, 0)),
            pl.BlockSpec((1, DH), lambda i: (0, 0)),
        ],
        out_shape=[
            jax.ShapeDtypeStruct((NPAD, DH), jnp.float32),
            jax.ShapeDtypeStruct((NPAD, DH), jnp.float32),
            jax.ShapeDtypeStruct((1, DH), jnp.float32),
        ],
    )(p0, p1, cnts, ones, w1t, w2t)


def _tc_head_body(p0_ref, p1_ref, p2_ref, esum_ref, wt0_ref, wt1_ref, wt2_ref,
                  bp_ref, out_ref):
    acc = (jnp.dot(p0_ref[...], wt0_ref[...], preferred_element_type=jnp.float32)
           + jnp.dot(p1_ref[...], wt1_ref[...], preferred_element_type=jnp.float32)
           + jnp.dot(p2_ref[...], wt2_ref[...], preferred_element_type=jnp.float32))
    out_ref[...] = jnp.maximum(acc / esum_ref[0, 0] + bp_ref[...], 0.0)


def _tc_head(p0, p1, p2, esum, wt0, wt1, wt2, bp):
    return pl.pallas_call(
        _tc_head_body,
        out_shape=jax.ShapeDtypeStruct((1, DF), jnp.float32),
    )(p0, p1, p2, esum, wt0, wt1, wt2, bp)


_sc_mesh = plsc.VectorSubcoreMesh(
    core_axis_name="c", subcore_axis_name="s", num_cores=NC, num_subcores=NS)


@functools.partial(
    pl.kernel,
    out_type=jax.ShapeDtypeStruct((NW * NPAD,), jnp.float32),
    mesh=_sc_mesh,
    compiler_params=pltpu.CompilerParams(needs_layout_passes=False),
    scratch_types=[
        pltpu.VMEM((KC,), jnp.int32),
        pltpu.VMEM((NPAD,), jnp.float32),
    ],
)
def _sc_count(src_hbm, outc_hbm, src_v, cnt_v):
    cid = lax.axis_index("c")
    sid = lax.axis_index("s")
    wid = cid * NS + sid

    zero16 = jnp.zeros((16,), jnp.float32)
    one16 = jnp.ones((16,), jnp.float32)

    def _zcnt(r, carry):
        cnt_v[pl.ds(r * 16, 16)] = zero16
        return carry

    lax.fori_loop(0, NPAD // 16, _zcnt, 0)

    ebase = wid * EPW

    def _block(j, carry):
        pltpu.sync_copy(src_hbm.at[pl.ds(ebase + j * KC, KC)], src_v)
        for q in range(KC // 16):
            plsc.addupdate_scatter(cnt_v, [src_v[pl.ds(q * 16, 16)]], one16)
        return carry

    lax.fori_loop(0, EPW // KC, _block, 0)
    pltpu.sync_copy(cnt_v, outc_hbm.at[pl.ds(wid * NPAD, NPAD)])


@functools.partial(
    pl.kernel,
    out_type=jax.ShapeDtypeStruct((NC * NPAD, DH), jnp.float32),
    mesh=_sc_mesh,
    compiler_params=pltpu.CompilerParams(needs_layout_passes=False),
    scratch_types=[
        pltpu.VMEM((K,), jnp.int32),
        pltpu.VMEM((K,), jnp.int32),
        pltpu.VMEM((K, DH), jnp.float32),
        pltpu.VMEM((K, DH), jnp.float32),
        pltpu.VMEM((K, DW), jnp.int32),
        pltpu.VMEM((K,), jnp.int32),
        pltpu.VMEM((K,), jnp.int32),
        pltpu.VMEM((K, DH), jnp.float32),
        pltpu.VMEM((K, DH), jnp.float32),
        pltpu.VMEM((K, DW), jnp.int32),
        pltpu.VMEM((K, DH), jnp.float32),
        pltpu.VMEM((K, DH), jnp.float32),
        pltpu.VMEM((K,), jnp.int32),
        pltpu.VMEM((K,), jnp.int32),
        pltpu.VMEM_SHARED((NPAD, DH), jnp.float32),
        pltpu.SemaphoreType.DMA,
        pltpu.SemaphoreType.DMA,
        pltpu.SemaphoreType.DMA,
        pltpu.SemaphoreType.DMA,
        pltpu.SemaphoreType.DMA,
        pltpu.SemaphoreType.DMA,
        pltpu.SemaphoreType.DMA,
        pltpu.SemaphoreType.DMA,
        pltpu.SemaphoreType.DMA,
        pltpu.SemaphoreType.DMA,
        pltpu.SemaphoreType.DMA,
        pltpu.SemaphoreType.DMA,
    ],
)
def _sc_edge(a_hbm, b_hbm, c_hbm, src_hbm, dst_hbm, out_hbm,
             src0, dst0, ra0, rb0, rc0, src1, dst1, ra1, rb1, rc1,
             ov0, ov1, sx0, sx1, acc_sh,
             sa0, sb0, sc0, sa1, sb1, sc1, si0, si1, sd0, sd1, ss0, ss1):
    cid = lax.axis_index("c")
    sid = lax.axis_index("s")
    wid = cid * NS + sid

    srcs = (src0, src1)
    dsts = (dst0, dst1)
    ras = (ra0, ra1)
    rbs = (rb0, rb1)
    rcs = (rc0, rc1)
    ovs = (ov0, ov1)
    sxs = (sx0, sx1)
    sss = (ss0, ss1)
    sas = (sa0, sa1)
    sbs = (sb0, sb1)
    scs = (sc0, sc1)
    sis = (si0, si1)
    sds = (sd0, sd1)

    zero16 = jnp.zeros((16,), jnp.float32)

    himask = jnp.full((16,), -65536, jnp.int32)

    def _zrow(r, carry):
        for c in range(DH // 16):
            ov0[r, pl.ds(c * 16, 16)] = zero16
        return carry

    lax.fori_loop(0, K, _zrow, 0)
    for z in range(RPT // K):
        pltpu.sync_copy(ov0, acc_sh.at[pl.ds(sid * RPT + z * K, K)])
    plsc.subcore_barrier()

    ebase = wid * EPW

    def _issue_idx(j, b):
        eb = ebase + j * K
        pltpu.async_copy(src_hbm.at[pl.ds(eb, K)], srcs[b], sis[b])
        pltpu.async_copy(dst_hbm.at[pl.ds(eb, K)], dsts[b], sds[b])

    def _wait_idx(b):
        pltpu.make_async_copy(src_hbm.at[pl.ds(0, K)], srcs[b], sis[b]).wait()
        pltpu.make_async_copy(dst_hbm.at[pl.ds(0, K)], dsts[b], sds[b]).wait()

    def _issue_rows(j, b):
        eb = ebase + j * K
        pltpu.async_copy(a_hbm.at[srcs[b]], ras[b], sas[b])
        pltpu.async_copy(b_hbm.at[dsts[b]], rbs[b], sbs[b])
        pltpu.async_copy(c_hbm.at[pl.ds(eb, K)], rcs[b], scs[b])

    def _copy_sidx(b):
        sxs[b][pl.ds(0, 16)] = srcs[b][pl.ds(0, 16)]
        sxs[b][pl.ds(16, 16)] = srcs[b][pl.ds(16, 16)]
        sxs[b][pl.ds(24, 16)] = srcs[b][pl.ds(24, 16)]

    for b in range(2):
        _issue_idx(b, b)
        _wait_idx(b)
        _copy_sidx(b)
        _issue_rows(b, b)

    def _round(g, carry):
        for b in range(2):
            j = 2 * g + b
            pltpu.make_async_copy(a_hbm.at[srcs[b]], ras[b], sas[b]).wait()
            pltpu.make_async_copy(b_hbm.at[dsts[b]], rbs[b], sbs[b]).wait()
            pltpu.make_async_copy(c_hbm.at[pl.ds(0, K)], rcs[b],
                                  scs[b]).wait()

            @pl.when(j + 2 < NBLK)
            def _():
                _issue_idx(j + 2, b)

            @pl.when(g >= 1)
            def _():
                pltpu.make_async_copy(ovs[b], acc_sh.at[sxs[b]],
                                      sss[b]).wait()

            ra, rb, rc, ov = ras[b], rbs[b], rcs[b], ovs[b]

            def _row(r, rc_):
                for c in range(DH // 32):
                    slo = pl.ds(c * 32, 16)
                    shi = pl.ds(c * 32 + 16, 16)
                    ci = rc[r, pl.ds(c * 16, 16)]
                    clo = plsc.bitcast(lax.shift_left(ci, 16), jnp.float32)
                    chi = plsc.bitcast(ci & himask, jnp.float32)
                    ov[r, slo] = jnp.maximum(ra[r, slo] + rb[r, slo] + clo,
                                             0.0)
                    ov[r, shi] = jnp.maximum(ra[r, shi] + rb[r, shi] + chi,
                                             0.0)
                return rc_

            lax.fori_loop(0, K, _row, 0)
            pltpu.async_copy(ov, acc_sh.at[sxs[b]], sss[b], add=True)

            @pl.when(j + 2 < NBLK)
            def _():
                _wait_idx(b)
                _copy_sidx(b)
                _issue_rows(j + 2, b)
        return carry

    lax.fori_loop(0, NBLK // 2, _round, 0)
    for b in range(2):
        pltpu.make_async_copy(ovs[b], acc_sh.at[sxs[b]], sss[b]).wait()
    plsc.subcore_barrier()

    for z in range(RPT // K):
        rs = sid * RPT + z * K
        pltpu.sync_copy(acc_sh.at[pl.ds(rs, K)], ov0)
        pltpu.sync_copy(ov0, out_hbm.at[pl.ds(cid * NPAD + rs, K)])


_MPERM = np.arange(DH).reshape(4, 2, 16).transpose(0, 2, 1).reshape(DH)


def kernel(child_feats, child_exists, edge_type_onehot, edge_feats,
           edge_indices, W_child, b_child, W_ne, b_ne, W_parent, b_parent):
    child = child_feats[0]
    exists = child_exists[0]
    ef = jnp.concatenate([edge_type_onehot[0], edge_feats[0]], axis=1)
    src = edge_indices[0, :, 0]
    dst = edge_indices[0, :, 1]

    wct = W_child.T
    w1t = W_ne[:, 0:DH].T
    w2t = W_ne[:, DH:2 * DH].T
    w3t = W_ne[:, 2 * DH:2 * DH + DEF].T[:, _MPERM]
    bc = b_child.reshape(1, DH)
    bne = b_ne.reshape(1, DH)[:, _MPERM]
    wt0 = W_parent[:, 0:DH].T
    wt1 = W_parent[:, DH:2 * DH].T
    wt2 = W_parent[:, 2 * DH:3 * DH].T
    bp = b_parent.reshape(1, DF)

    a0, b0, psum0, esum = _tc_pre(child, exists, wct, bc, w1t, w2t)
    c = _tc_edgeproj(ef, w3t, bne)

    ci = lax.bitcast_convert_type(c.reshape(E, DW, 2), jnp.int32)
    ones_nw = jnp.ones((NW, 1), jnp.float32)
    cnts = _sc_count(src).reshape(NW, NPAD)

    sums1 = _sc_edge(a0, b0, ci, src, dst)
    a1, b1, psum1 = _tc_mid(sums1[0:NPAD], sums1[NPAD:2 * NPAD],
                            cnts, ones_nw, w1t, w2t)

    sums2 = _sc_edge(a1, b1, ci, src, dst)
    _, _, psum2 = _tc_mid(sums2[0:NPAD], sums2[NPAD:2 * NPAD],
                          cnts, ones_nw, w1t, w2t)

    return _tc_head(psum0, psum1, psum2, esum, wt0, wt1, wt2, bp)


# speedup vs baseline: 1.1593x; 1.1593x over previous
import functools

import numpy as np

import jax
import jax.numpy as jnp
from jax import lax
from jax.experimental import pallas as pl
from jax.experimental.pallas import tpu as pltpu
from jax.experimental.pallas import tpu_sc as plsc

N = 10000
E = 320000
DF = 128
DH = 128
DEF = 20
ACC_W = 144

NC, NS = 2, 16
NW = NC * NS
EPW = E // NW
K = 40
NBLK = EPW // K
KC = 80
DW = 64
NPAD = 10240
RPT = NPAD // NS
RZC = 128


def _tc_pre_body(child_ref, exists_ref, wct_ref, bc_ref, w1t_ref, w2t_ref,
                 a_ref, b_ref, psum_ref, esum_ref):
    i = pl.program_id(0)
    x = child_ref[...]
    cf = jnp.maximum(
        jnp.dot(x, wct_ref[...], preferred_element_type=jnp.float32)
        + bc_ref[...], 0.0) * exists_ref[...]
    a_ref[...] = jnp.dot(cf, w1t_ref[...], preferred_element_type=jnp.float32)
    b_ref[...] = jnp.dot(cf, w2t_ref[...], preferred_element_type=jnp.float32)
    ps = jnp.sum(cf, axis=0, keepdims=True)
    es = jnp.sum(exists_ref[...], axis=0, keepdims=True)

    @pl.when(i == 0)
    def _():
        psum_ref[...] = ps
        esum_ref[...] = es

    @pl.when(i > 0)
    def _():
        psum_ref[...] += ps
        esum_ref[...] += es


def _tc_pre(child, exists, wct, bc, w1t, w2t):
    blk = 2000
    grid = (N // blk,)
    return pl.pallas_call(
        _tc_pre_body,
        grid=grid,
        in_specs=[
            pl.BlockSpec((blk, DF), lambda i: (i, 0)),
            pl.BlockSpec((blk, 1), lambda i: (i, 0)),
            pl.BlockSpec((DF, DH), lambda i: (0, 0)),
            pl.BlockSpec((1, DH), lambda i: (0, 0)),
            pl.BlockSpec((DH, DH), lambda i: (0, 0)),
            pl.BlockSpec((DH, DH), lambda i: (0, 0)),
        ],
        out_specs=[
            pl.BlockSpec((blk, DH), lambda i: (i, 0)),
            pl.BlockSpec((blk, DH), lambda i: (i, 0)),
            pl.BlockSpec((1, DH), lambda i: (0, 0)),
            pl.BlockSpec((1, 1), lambda i: (0, 0)),
        ],
        out_shape=[
            jax.ShapeDtypeStruct((N, DH), jnp.float32),
            jax.ShapeDtypeStruct((N, DH), jnp.float32),
            jax.ShapeDtypeStruct((1, DH), jnp.float32),
            jax.ShapeDtypeStruct((1, 1), jnp.float32),
        ],
    )(child, exists, wct, bc, w1t, w2t)


def _tc_edgeproj_body(ef_ref, w3t_ref, bne_ref, c_ref):
    c_ref[...] = (
        jnp.dot(ef_ref[...], w3t_ref[...], preferred_element_type=jnp.float32)
        + bne_ref[...]).astype(jnp.bfloat16)


def _tc_edgeproj(ef, w3t, bne):
    blk = 8000
    return pl.pallas_call(
        _tc_edgeproj_body,
        grid=(E // blk,),
        in_specs=[
            pl.BlockSpec((blk, DEF), lambda i: (i, 0)),
            pl.BlockSpec((DEF, DH), lambda i: (0, 0)),
            pl.BlockSpec((1, DH), lambda i: (0, 0)),
        ],
        out_specs=pl.BlockSpec((blk, DH), lambda i: (i, 0)),
        out_shape=jax.ShapeDtypeStruct((E, DH), jnp.bfloat16),
    )(ef, w3t, bne)


def _tc_mid_body(p0_ref, p1_ref, cnt_ref, ones_ref, w1t_ref, w2t_ref,
                 a_ref, b_ref, psum_ref):
    i = pl.program_id(0)
    counts = lax.dot_general(cnt_ref[...], ones_ref[...],
                             (((0,), (0,)), ((), ())),
                             preferred_element_type=jnp.float32)
    cf = (p0_ref[...] + p1_ref[...]) / jnp.maximum(counts, 1.0)
    a_ref[...] = jnp.dot(cf, w1t_ref[...], preferred_element_type=jnp.float32)
    b_ref[...] = jnp.dot(cf, w2t_ref[...], preferred_element_type=jnp.float32)
    ps = jnp.sum(cf, axis=0, keepdims=True)

    @pl.when(i == 0)
    def _():
        psum_ref[...] = ps

    @pl.when(i > 0)
    def _():
        psum_ref[...] += ps


def _tc_mid(p0, p1, cnts, ones, w1t, w2t):
    blk = 2048
    return pl.pallas_call(
        _tc_mid_body,
        grid=(NPAD // blk,),
        in_specs=[
            pl.BlockSpec((blk, DH), lambda i: (i, 0)),
            pl.BlockSpec((blk, DH), lambda i: (i, 0)),
            pl.BlockSpec((NW, blk), lambda i: (0, i)),
            pl.BlockSpec((NW, 1), lambda i: (0, 0)),
            pl.BlockSpec((DH, DH), lambda i: (0, 0)),
            pl.BlockSpec((DH, DH), lambda i: (0, 0)),
        ],
        out_specs=[
            pl.BlockSpec((blk, DH), lambda i: (i, 0)),
            pl.BlockSpec((blk, DH), lambda i: (i, 0)),
            pl.BlockSpec((1, DH), lambda i: (0, 0)),
        ],
        out_shape=[
            jax.ShapeDtypeStruct((NPAD, DH), jnp.float32),
            jax.ShapeDtypeStruct((NPAD, DH), jnp.float32),
            jax.ShapeDtypeStruct((1, DH), jnp.float32),
        ],
    )(p0, p1, cnts, ones, w1t, w2t)


def _tc_head_body(p0_ref, p1_ref, p2_ref, esum_ref, wt0_ref, wt1_ref, wt2_ref,
                  bp_ref, out_ref):
    acc = (jnp.dot(p0_ref[...], wt0_ref[...], preferred_element_type=jnp.float32)
           + jnp.dot(p1_ref[...], wt1_ref[...], preferred_element_type=jnp.float32)
           + jnp.dot(p2_ref[...], wt2_ref[...], preferred_element_type=jnp.float32))
    out_ref[...] = jnp.maximum(acc / esum_ref[0, 0] + bp_ref[...], 0.0)


def _tc_head(p0, p1, p2, esum, wt0, wt1, wt2, bp):
    return pl.pallas_call(
        _tc_head_body,
        out_shape=jax.ShapeDtypeStruct((1, DF), jnp.float32),
    )(p0, p1, p2, esum, wt0, wt1, wt2, bp)


_sc_mesh = plsc.VectorSubcoreMesh(
    core_axis_name="c", subcore_axis_name="s", num_cores=NC, num_subcores=NS)


@functools.partial(
    pl.kernel,
    out_type=jax.ShapeDtypeStruct((NW * NPAD,), jnp.float32),
    mesh=_sc_mesh,
    compiler_params=pltpu.CompilerParams(needs_layout_passes=False),
    scratch_types=[
        pltpu.VMEM((KC,), jnp.int32),
        pltpu.VMEM((NPAD,), jnp.float32),
    ],
)
def _sc_count(src_hbm, outc_hbm, src_v, cnt_v):
    cid = lax.axis_index("c")
    sid = lax.axis_index("s")
    wid = cid * NS + sid

    zero16 = jnp.zeros((16,), jnp.float32)
    one16 = jnp.ones((16,), jnp.float32)

    def _zcnt(r, carry):
        cnt_v[pl.ds(r * 16, 16)] = zero16
        return carry

    lax.fori_loop(0, NPAD // 16, _zcnt, 0)

    ebase = wid * EPW

    def _block(j, carry):
        pltpu.sync_copy(src_hbm.at[pl.ds(ebase + j * KC, KC)], src_v)
        for q in range(KC // 16):
            plsc.addupdate_scatter(cnt_v, [src_v[pl.ds(q * 16, 16)]], one16)
        return carry

    lax.fori_loop(0, EPW // KC, _block, 0)
    pltpu.sync_copy(cnt_v, outc_hbm.at[pl.ds(wid * NPAD, NPAD)])


@functools.partial(
    pl.kernel,
    out_type=jax.ShapeDtypeStruct((NC * NPAD, DH), jnp.float32),
    mesh=_sc_mesh,
    compiler_params=pltpu.CompilerParams(needs_layout_passes=False),
    scratch_types=[
        pltpu.VMEM((K,), jnp.int32),
        pltpu.VMEM((K,), jnp.int32),
        pltpu.VMEM((K, DH), jnp.float32),
        pltpu.VMEM((K, DH), jnp.float32),
        pltpu.VMEM((K, DH), jnp.bfloat16),
        pltpu.VMEM((K,), jnp.int32),
        pltpu.VMEM((K,), jnp.int32),
        pltpu.VMEM((K, DH), jnp.float32),
        pltpu.VMEM((K, DH), jnp.float32),
        pltpu.VMEM((K, DH), jnp.bfloat16),
        pltpu.VMEM((K, DH), jnp.float32),
        pltpu.VMEM((K, DH), jnp.float32),
        pltpu.VMEM((K,), jnp.int32),
        pltpu.VMEM((K,), jnp.int32),
        pltpu.VMEM_SHARED((NPAD, DH), jnp.float32),
        pltpu.SemaphoreType.DMA,
        pltpu.SemaphoreType.DMA,
        pltpu.SemaphoreType.DMA,
        pltpu.SemaphoreType.DMA,
        pltpu.SemaphoreType.DMA,
        pltpu.SemaphoreType.DMA,
        pltpu.SemaphoreType.DMA,
        pltpu.SemaphoreType.DMA,
        pltpu.SemaphoreType.DMA,
        pltpu.SemaphoreType.DMA,
        pltpu.SemaphoreType.DMA,
        pltpu.SemaphoreType.DMA,
    ],
)
def _sc_edge(a_hbm, b_hbm, c_hbm, src_hbm, dst_hbm, out_hbm,
             src0, dst0, ra0, rb0, rc0, src1, dst1, ra1, rb1, rc1,
             ov0, ov1, sx0, sx1, acc_sh,
             sa0, sb0, sc0, sa1, sb1, sc1, si0, si1, sd0, sd1, ss0, ss1):
    cid = lax.axis_index("c")
    sid = lax.axis_index("s")
    wid = cid * NS + sid

    srcs = (src0, src1)
    dsts = (dst0, dst1)
    ras = (ra0, ra1)
    rbs = (rb0, rb1)
    rcs = (rc0, rc1)
    ovs = (ov0, ov1)
    sxs = (sx0, sx1)
    sss = (ss0, ss1)
    sas = (sa0, sa1)
    sbs = (sb0, sb1)
    scs = (sc0, sc1)
    sis = (si0, si1)
    sds = (sd0, sd1)

    zero16 = jnp.zeros((16,), jnp.float32)

    himask = jnp.full((16,), -65536, jnp.int32)

    def _zrow(r, carry):
        for c in range(DH // 16):
            ov0[r, pl.ds(c * 16, 16)] = zero16
        return carry

    lax.fori_loop(0, K, _zrow, 0)
    for z in range(RPT // K):
        pltpu.sync_copy(ov0, acc_sh.at[pl.ds(sid * RPT + z * K, K)])
    plsc.subcore_barrier()

    ebase = wid * EPW

    def _issue_idx(j, b):
        eb = ebase + j * K
        pltpu.async_copy(src_hbm.at[pl.ds(eb, K)], srcs[b], sis[b])
        pltpu.async_copy(dst_hbm.at[pl.ds(eb, K)], dsts[b], sds[b])

    def _wait_idx(b):
        pltpu.make_async_copy(src_hbm.at[pl.ds(0, K)], srcs[b], sis[b]).wait()
        pltpu.make_async_copy(dst_hbm.at[pl.ds(0, K)], dsts[b], sds[b]).wait()

    def _issue_rows(j, b):
        eb = ebase + j * K
        pltpu.async_copy(a_hbm.at[srcs[b]], ras[b], sas[b])
        pltpu.async_copy(b_hbm.at[dsts[b]], rbs[b], sbs[b])
        pltpu.async_copy(c_hbm.at[pl.ds(eb, K)], rcs[b], scs[b])

    def _copy_sidx(b):
        sxs[b][pl.ds(0, 16)] = srcs[b][pl.ds(0, 16)]
        sxs[b][pl.ds(16, 16)] = srcs[b][pl.ds(16, 16)]
        sxs[b][pl.ds(24, 16)] = srcs[b][pl.ds(24, 16)]

    for b in range(2):
        _issue_idx(b, b)
        _wait_idx(b)
        _copy_sidx(b)
        _issue_rows(b, b)

    def _round(g, carry):
        for b in range(2):
            j = 2 * g + b
            pltpu.make_async_copy(a_hbm.at[srcs[b]], ras[b], sas[b]).wait()
            pltpu.make_async_copy(b_hbm.at[dsts[b]], rbs[b], sbs[b]).wait()
            pltpu.make_async_copy(c_hbm.at[pl.ds(0, K)], rcs[b],
                                  scs[b]).wait()

            @pl.when(j + 2 < NBLK)
            def _():
                _issue_idx(j + 2, b)

            @pl.when(g >= 1)
            def _():
                pltpu.make_async_copy(ovs[b], acc_sh.at[sxs[b]],
                                      sss[b]).wait()

            ra, rb, rc, ov = ras[b], rbs[b], rcs[b], ovs[b]

            def _row(r, rc_):
                for c in range(DH // 32):
                    slo = pl.ds(c * 32, 16)
                    shi = pl.ds(c * 32 + 16, 16)
                    ci = plsc.bitcast(rc[r, pl.ds(c * 32, 32)], jnp.int32)
                    clo = plsc.bitcast(lax.shift_left(ci, 16), jnp.float32)
                    chi = plsc.bitcast(ci & himask, jnp.float32)
                    ov[r, slo] = jnp.maximum(ra[r, slo] + rb[r, slo] + clo,
                                             0.0)
                    ov[r, shi] = jnp.maximum(ra[r, shi] + rb[r, shi] + chi,
                                             0.0)
                return rc_

            lax.fori_loop(0, K, _row, 0)
            pltpu.async_copy(ov, acc_sh.at[sxs[b]], sss[b], add=True)

            @pl.when(j + 2 < NBLK)
            def _():
                _wait_idx(b)
                _copy_sidx(b)
                _issue_rows(j + 2, b)
        return carry

    lax.fori_loop(0, NBLK // 2, _round, 0)
    for b in range(2):
        pltpu.make_async_copy(ovs[b], acc_sh.at[sxs[b]], sss[b]).wait()
    plsc.subcore_barrier()

    for z in range(RPT // K):
        rs = sid * RPT + z * K
        pltpu.sync_copy(acc_sh.at[pl.ds(rs, K)], ov0)
        pltpu.sync_copy(ov0, out_hbm.at[pl.ds(cid * NPAD + rs, K)])


_MPERM = np.arange(DH).reshape(4, 2, 16).transpose(0, 2, 1).reshape(DH)


def kernel(child_feats, child_exists, edge_type_onehot, edge_feats,
           edge_indices, W_child, b_child, W_ne, b_ne, W_parent, b_parent):
    child = child_feats[0]
    exists = child_exists[0]
    ef = jnp.concatenate([edge_type_onehot[0], edge_feats[0]], axis=1)
    src = edge_indices[0, :, 0]
    dst = edge_indices[0, :, 1]

    wct = W_child.T
    w1t = W_ne[:, 0:DH].T
    w2t = W_ne[:, DH:2 * DH].T
    w3t = W_ne[:, 2 * DH:2 * DH + DEF].T[:, _MPERM]
    bc = b_child.reshape(1, DH)
    bne = b_ne.reshape(1, DH)[:, _MPERM]
    wt0 = W_parent[:, 0:DH].T
    wt1 = W_parent[:, DH:2 * DH].T
    wt2 = W_parent[:, 2 * DH:3 * DH].T
    bp = b_parent.reshape(1, DF)

    a0, b0, psum0, esum = _tc_pre(child, exists, wct, bc, w1t, w2t)
    c = _tc_edgeproj(ef, w3t, bne)

    ones_nw = jnp.ones((NW, 1), jnp.float32)
    cnts = _sc_count(src).reshape(NW, NPAD)

    sums1 = _sc_edge(a0, b0, c, src, dst)
    a1, b1, psum1 = _tc_mid(sums1[0:NPAD], sums1[NPAD:2 * NPAD],
                            cnts, ones_nw, w1t, w2t)

    sums2 = _sc_edge(a1, b1, c, src, dst)
    _, _, psum2 = _tc_mid(sums2[0:NPAD], sums2[NPAD:2 * NPAD],
                          cnts, ones_nw, w1t, w2t)

    return _tc_head(psum0, psum1, psum2, esum, wt0, wt1, wt2, bp)


# speedup vs baseline: 1.9097x; 1.6472x over previous
import functools

import jax
import jax.numpy as jnp
from jax import lax
from jax.experimental import pallas as pl
from jax.experimental.pallas import tpu as pltpu
from jax.experimental.pallas import tpu_sc as plsc

N = 10000
E = 320000
DF = 128
DH = 128
DEF = 20
ACC_W = 144

NC, NS = 2, 16
NW = NC * NS
EPW = E // NW
K = 40
NBLK = EPW // K
KC = 80
NPAD = 10240
RPT = NPAD // NS
RZC = 128


def _tc_pre_body(child_ref, exists_ref, wct_ref, bc_ref, w1t_ref, w2t_ref,
                 a_ref, b_ref, psum_ref, esum_ref):
    i = pl.program_id(0)
    x = child_ref[...]
    cf = jnp.maximum(
        jnp.dot(x, wct_ref[...], preferred_element_type=jnp.float32)
        + bc_ref[...], 0.0) * exists_ref[...]
    a_ref[...] = jnp.dot(cf, w1t_ref[...], preferred_element_type=jnp.float32)
    b_ref[...] = jnp.dot(cf, w2t_ref[...], preferred_element_type=jnp.float32)
    ps = jnp.sum(cf, axis=0, keepdims=True)
    es = jnp.sum(exists_ref[...], axis=0, keepdims=True)

    @pl.when(i == 0)
    def _():
        psum_ref[...] = ps
        esum_ref[...] = es

    @pl.when(i > 0)
    def _():
        psum_ref[...] += ps
        esum_ref[...] += es


def _tc_pre(child, exists, wct, bc, w1t, w2t):
    blk = 2000
    grid = (N // blk,)
    return pl.pallas_call(
        _tc_pre_body,
        grid=grid,
        in_specs=[
            pl.BlockSpec((blk, DF), lambda i: (i, 0)),
            pl.BlockSpec((blk, 1), lambda i: (i, 0)),
            pl.BlockSpec((DF, DH), lambda i: (0, 0)),
            pl.BlockSpec((1, DH), lambda i: (0, 0)),
            pl.BlockSpec((DH, DH), lambda i: (0, 0)),
            pl.BlockSpec((DH, DH), lambda i: (0, 0)),
        ],
        out_specs=[
            pl.BlockSpec((blk, DH), lambda i: (i, 0)),
            pl.BlockSpec((blk, DH), lambda i: (i, 0)),
            pl.BlockSpec((1, DH), lambda i: (0, 0)),
            pl.BlockSpec((1, 1), lambda i: (0, 0)),
        ],
        out_shape=[
            jax.ShapeDtypeStruct((N, DH), jnp.float32),
            jax.ShapeDtypeStruct((N, DH), jnp.float32),
            jax.ShapeDtypeStruct((1, DH), jnp.float32),
            jax.ShapeDtypeStruct((1, 1), jnp.float32),
        ],
    )(child, exists, wct, bc, w1t, w2t)


def _tc_edgeproj_body(ef_ref, w3t_ref, bne_ref, c_ref):
    c_ref[...] = (
        jnp.dot(ef_ref[...], w3t_ref[...], preferred_element_type=jnp.float32)
        + bne_ref[...])


def _tc_edgeproj(ef, w3t, bne):
    blk = 8000
    return pl.pallas_call(
        _tc_edgeproj_body,
        grid=(E // blk,),
        in_specs=[
            pl.BlockSpec((blk, DEF), lambda i: (i, 0)),
            pl.BlockSpec((DEF, DH), lambda i: (0, 0)),
            pl.BlockSpec((1, DH), lambda i: (0, 0)),
        ],
        out_specs=pl.BlockSpec((blk, DH), lambda i: (i, 0)),
        out_shape=jax.ShapeDtypeStruct((E, DH), jnp.float32),
    )(ef, w3t, bne)


def _tc_mid_body(p0_ref, p1_ref, cnt_ref, ones_ref, w1t_ref, w2t_ref,
                 a_ref, b_ref, psum_ref):
    i = pl.program_id(0)
    counts = lax.dot_general(cnt_ref[...], ones_ref[...],
                             (((0,), (0,)), ((), ())),
                             preferred_element_type=jnp.float32)
    cf = (p0_ref[...] + p1_ref[...]) / jnp.maximum(counts, 1.0)
    a_ref[...] = jnp.dot(cf, w1t_ref[...], preferred_element_type=jnp.float32)
    b_ref[...] = jnp.dot(cf, w2t_ref[...], preferred_element_type=jnp.float32)
    ps = jnp.sum(cf, axis=0, keepdims=True)

    @pl.when(i == 0)
    def _():
        psum_ref[...] = ps

    @pl.when(i > 0)
    def _():
        psum_ref[...] += ps


def _tc_mid(p0, p1, cnts, ones, w1t, w2t):
    blk = 2048
    return pl.pallas_call(
        _tc_mid_body,
        grid=(NPAD // blk,),
        in_specs=[
            pl.BlockSpec((blk, DH), lambda i: (i, 0)),
            pl.BlockSpec((blk, DH), lambda i: (i, 0)),
            pl.BlockSpec((NW, blk), lambda i: (0, i)),
            pl.BlockSpec((NW, 1), lambda i: (0, 0)),
            pl.BlockSpec((DH, DH), lambda i: (0, 0)),
            pl.BlockSpec((DH, DH), lambda i: (0, 0)),
        ],
        out_specs=[
            pl.BlockSpec((blk, DH), lambda i: (i, 0)),
            pl.BlockSpec((blk, DH), lambda i: (i, 0)),
            pl.BlockSpec((1, DH), lambda i: (0, 0)),
        ],
        out_shape=[
            jax.ShapeDtypeStruct((NPAD, DH), jnp.float32),
            jax.ShapeDtypeStruct((NPAD, DH), jnp.float32),
            jax.ShapeDtypeStruct((1, DH), jnp.float32),
        ],
    )(p0, p1, cnts, ones, w1t, w2t)


def _tc_head_body(p0_ref, p1_ref, p2_ref, esum_ref, wt0_ref, wt1_ref, wt2_ref,
                  bp_ref, out_ref):
    acc = (jnp.dot(p0_ref[...], wt0_ref[...], preferred_element_type=jnp.float32)
           + jnp.dot(p1_ref[...], wt1_ref[...], preferred_element_type=jnp.float32)
           + jnp.dot(p2_ref[...], wt2_ref[...], preferred_element_type=jnp.float32))
    out_ref[...] = jnp.maximum(acc / esum_ref[0, 0] + bp_ref[...], 0.0)


def _tc_head(p0, p1, p2, esum, wt0, wt1, wt2, bp):
    return pl.pallas_call(
        _tc_head_body,
        out_shape=jax.ShapeDtypeStruct((1, DF), jnp.float32),
    )(p0, p1, p2, esum, wt0, wt1, wt2, bp)


_sc_mesh = plsc.VectorSubcoreMesh(
    core_axis_name="c", subcore_axis_name="s", num_cores=NC, num_subcores=NS)


@functools.partial(
    pl.kernel,
    out_type=jax.ShapeDtypeStruct((NW * NPAD,), jnp.float32),
    mesh=_sc_mesh,
    compiler_params=pltpu.CompilerParams(needs_layout_passes=False),
    scratch_types=[
        pltpu.VMEM((KC,), jnp.int32),
        pltpu.VMEM((NPAD,), jnp.float32),
    ],
)
def _sc_count(src_hbm, outc_hbm, src_v, cnt_v):
    cid = lax.axis_index("c")
    sid = lax.axis_index("s")
    wid = cid * NS + sid

    zero16 = jnp.zeros((16,), jnp.float32)
    one16 = jnp.ones((16,), jnp.float32)

    def _zcnt(r, carry):
        cnt_v[pl.ds(r * 16, 16)] = zero16
        return carry

    lax.fori_loop(0, NPAD // 16, _zcnt, 0)

    ebase = wid * EPW

    def _block(j, carry):
        pltpu.sync_copy(src_hbm.at[pl.ds(ebase + j * KC, KC)], src_v)
        for q in range(KC // 16):
            plsc.addupdate_scatter(cnt_v, [src_v[pl.ds(q * 16, 16)]], one16)
        return carry

    lax.fori_loop(0, EPW // KC, _block, 0)
    pltpu.sync_copy(cnt_v, outc_hbm.at[pl.ds(wid * NPAD, NPAD)])


@functools.partial(
    pl.kernel,
    out_type=jax.ShapeDtypeStruct((NC * NPAD, DH), jnp.float32),
    mesh=_sc_mesh,
    compiler_params=pltpu.CompilerParams(needs_layout_passes=False),
    scratch_types=[
        pltpu.VMEM((K,), jnp.int32),
        pltpu.VMEM((K,), jnp.int32),
        pltpu.VMEM((K, DH), jnp.float32),
        pltpu.VMEM((K, DH), jnp.float32),
        pltpu.VMEM((K, DH), jnp.float32),
        pltpu.VMEM((K,), jnp.int32),
        pltpu.VMEM((K,), jnp.int32),
        pltpu.VMEM((K, DH), jnp.float32),
        pltpu.VMEM((K, DH), jnp.float32),
        pltpu.VMEM((K, DH), jnp.float32),
        pltpu.VMEM((K, DH), jnp.float32),
        pltpu.VMEM((K, DH), jnp.float32),
        pltpu.VMEM((K,), jnp.int32),
        pltpu.VMEM((K,), jnp.int32),
        pltpu.VMEM_SHARED((NPAD, DH), jnp.float32),
        pltpu.SemaphoreType.DMA,
        pltpu.SemaphoreType.DMA,
        pltpu.SemaphoreType.DMA,
        pltpu.SemaphoreType.DMA,
        pltpu.SemaphoreType.DMA,
        pltpu.SemaphoreType.DMA,
        pltpu.SemaphoreType.DMA,
        pltpu.SemaphoreType.DMA,
        pltpu.SemaphoreType.DMA,
        pltpu.SemaphoreType.DMA,
        pltpu.SemaphoreType.DMA,
        pltpu.SemaphoreType.DMA,
    ],
)
def _sc_edge(a_hbm, b_hbm, c_hbm, src_hbm, dst_hbm, out_hbm,
             src0, dst0, ra0, rb0, rc0, src1, dst1, ra1, rb1, rc1,
             ov0, ov1, sx0, sx1, acc_sh,
             sa0, sb0, sc0, sa1, sb1, sc1, si0, si1, sd0, sd1, ss0, ss1):
    cid = lax.axis_index("c")
    sid = lax.axis_index("s")
    wid = cid * NS + sid

    srcs = (src0, src1)
    dsts = (dst0, dst1)
    ras = (ra0, ra1)
    rbs = (rb0, rb1)
    rcs = (rc0, rc1)
    ovs = (ov0, ov1)
    sxs = (sx0, sx1)
    sss = (ss0, ss1)
    sas = (sa0, sa1)
    sbs = (sb0, sb1)
    scs = (sc0, sc1)
    sis = (si0, si1)
    sds = (sd0, sd1)

    zero16 = jnp.zeros((16,), jnp.float32)

    def _zrow(r, carry):
        for c in range(DH // 16):
            ov0[r, pl.ds(c * 16, 16)] = zero16
        return carry

    lax.fori_loop(0, K, _zrow, 0)
    for z in range(RPT // K):
        pltpu.sync_copy(ov0, acc_sh.at[pl.ds(sid * RPT + z * K, K)])
    plsc.subcore_barrier()

    ebase = wid * EPW

    def _issue_idx(j, b):
        eb = ebase + j * K
        pltpu.async_copy(src_hbm.at[pl.ds(eb, K)], srcs[b], sis[b])
        pltpu.async_copy(dst_hbm.at[pl.ds(eb, K)], dsts[b], sds[b])

    def _wait_idx(b):
        pltpu.make_async_copy(src_hbm.at[pl.ds(0, K)], srcs[b], sis[b]).wait()
        pltpu.make_async_copy(dst_hbm.at[pl.ds(0, K)], dsts[b], sds[b]).wait()

    def _issue_rows(j, b):
        eb = ebase + j * K
        pltpu.async_copy(a_hbm.at[srcs[b]], ras[b], sas[b])
        pltpu.async_copy(b_hbm.at[dsts[b]], rbs[b], sbs[b])
        pltpu.async_copy(c_hbm.at[pl.ds(eb, K)], rcs[b], scs[b])

    def _copy_sidx(b):
        sxs[b][pl.ds(0, 16)] = srcs[b][pl.ds(0, 16)]
        sxs[b][pl.ds(16, 16)] = srcs[b][pl.ds(16, 16)]
        sxs[b][pl.ds(24, 16)] = srcs[b][pl.ds(24, 16)]

    for b in range(2):
        _issue_idx(b, b)
        _wait_idx(b)
        _copy_sidx(b)
        _issue_rows(b, b)

    def _round(g, carry):
        for b in range(2):
            j = 2 * g + b
            pltpu.make_async_copy(a_hbm.at[srcs[b]], ras[b], sas[b]).wait()
            pltpu.make_async_copy(b_hbm.at[dsts[b]], rbs[b], sbs[b]).wait()
            pltpu.make_async_copy(c_hbm.at[pl.ds(0, K)], rcs[b],
                                  scs[b]).wait()

            @pl.when(j + 2 < NBLK)
            def _():
                _issue_idx(j + 2, b)

            @pl.when(g >= 1)
            def _():
                pltpu.make_async_copy(ovs[b], acc_sh.at[sxs[b]],
                                      sss[b]).wait()

            ra, rb, rc, ov = ras[b], rbs[b], rcs[b], ovs[b]

            def _row(r, rc_):
                for c in range(DH // 16):
                    s = pl.ds(c * 16, 16)
                    ov[r, s] = jnp.maximum(
                        ra[r, s] + rb[r, s] + rc[r, s], 0.0)
                return rc_

            lax.fori_loop(0, K, _row, 0)
            pltpu.async_copy(ov, acc_sh.at[sxs[b]], sss[b], add=True)

            @pl.when(j + 2 < NBLK)
            def _():
                _wait_idx(b)
                _copy_sidx(b)
                _issue_rows(j + 2, b)
        return carry

    lax.fori_loop(0, NBLK // 2, _round, 0)
    for b in range(2):
        pltpu.make_async_copy(ovs[b], acc_sh.at[sxs[b]], sss[b]).wait()
    plsc.subcore_barrier()

    for z in range(RPT // K):
        rs = sid * RPT + z * K
        pltpu.sync_copy(acc_sh.at[pl.ds(rs, K)], ov0)
        pltpu.sync_copy(ov0, out_hbm.at[pl.ds(cid * NPAD + rs, K)])


def kernel(child_feats, child_exists, edge_type_onehot, edge_feats,
           edge_indices, W_child, b_child, W_ne, b_ne, W_parent, b_parent):
    child = child_feats[0]
    exists = child_exists[0]
    ef = jnp.concatenate([edge_type_onehot[0], edge_feats[0]], axis=1)
    src = edge_indices[0, :, 0]
    dst = edge_indices[0, :, 1]

    wct = W_child.T
    w1t = W_ne[:, 0:DH].T
    w2t = W_ne[:, DH:2 * DH].T
    w3t = W_ne[:, 2 * DH:2 * DH + DEF].T
    bc = b_child.reshape(1, DH)
    bne = b_ne.reshape(1, DH)
    wt0 = W_parent[:, 0:DH].T
    wt1 = W_parent[:, DH:2 * DH].T
    wt2 = W_parent[:, 2 * DH:3 * DH].T
    bp = b_parent.reshape(1, DF)

    a0, b0, psum0, esum = _tc_pre(child, exists, wct, bc, w1t, w2t)
    c = _tc_edgeproj(ef, w3t, bne)

    ones_nw = jnp.ones((NW, 1), jnp.float32)
    cnts = _sc_count(src).reshape(NW, NPAD)

    sums1 = _sc_edge(a0, b0, c, src, dst)
    a1, b1, psum1 = _tc_mid(sums1[0:NPAD], sums1[NPAD:2 * NPAD],
                            cnts, ones_nw, w1t, w2t)

    sums2 = _sc_edge(a1, b1, c, src, dst)
    _, _, psum2 = _tc_mid(sums2[0:NPAD], sums2[NPAD:2 * NPAD],
                          cnts, ones_nw, w1t, w2t)

    return _tc_head(psum0, psum1, psum2, esum, wt0, wt1, wt2, bp)


# speedup vs baseline: 1.9108x; 1.0006x over previous
import functools

import jax
import jax.numpy as jnp
from jax import lax
from jax.experimental import pallas as pl
from jax.experimental.pallas import tpu as pltpu
from jax.experimental.pallas import tpu_sc as plsc

N = 10000
E = 320000
DF = 128
DH = 128
DEF = 20
NC, NS = 2, 16
NW = NC * NS
EPW = E // NW
K = 40
NBLK = EPW // K
KC = 80
NPAD = 10240
RPT = NPAD // NS


def _tc_pre_body(child_ref, exists_ref, wct_ref, bc_ref, w1t_ref, w2t_ref,
                 a_ref, b_ref, psum_ref, esum_ref):
    i = pl.program_id(0)
    x = child_ref[...]
    cf = jnp.maximum(
        jnp.dot(x, wct_ref[...], preferred_element_type=jnp.float32)
        + bc_ref[...], 0.0) * exists_ref[...]
    a_ref[...] = jnp.dot(cf, w1t_ref[...], preferred_element_type=jnp.float32)
    b_ref[...] = jnp.dot(cf, w2t_ref[...], preferred_element_type=jnp.float32)
    ps = jnp.sum(cf, axis=0, keepdims=True)
    es = jnp.sum(exists_ref[...], axis=0, keepdims=True)

    @pl.when(i == 0)
    def _():
        psum_ref[...] = ps
        esum_ref[...] = es

    @pl.when(i > 0)
    def _():
        psum_ref[...] += ps
        esum_ref[...] += es


def _tc_pre(child, exists, wct, bc, w1t, w2t):
    blk = 2000
    grid = (N // blk,)
    return pl.pallas_call(
        _tc_pre_body,
        grid=grid,
        in_specs=[
            pl.BlockSpec((blk, DF), lambda i: (i, 0)),
            pl.BlockSpec((blk, 1), lambda i: (i, 0)),
            pl.BlockSpec((DF, DH), lambda i: (0, 0)),
            pl.BlockSpec((1, DH), lambda i: (0, 0)),
            pl.BlockSpec((DH, DH), lambda i: (0, 0)),
            pl.BlockSpec((DH, DH), lambda i: (0, 0)),
        ],
        out_specs=[
            pl.BlockSpec((blk, DH), lambda i: (i, 0)),
            pl.BlockSpec((blk, DH), lambda i: (i, 0)),
            pl.BlockSpec((1, DH), lambda i: (0, 0)),
            pl.BlockSpec((1, 1), lambda i: (0, 0)),
        ],
        out_shape=[
            jax.ShapeDtypeStruct((N, DH), jnp.float32),
            jax.ShapeDtypeStruct((N, DH), jnp.float32),
            jax.ShapeDtypeStruct((1, DH), jnp.float32),
            jax.ShapeDtypeStruct((1, 1), jnp.float32),
        ],
    )(child, exists, wct, bc, w1t, w2t)


def _tc_edgeproj_body(ef_ref, w3t_ref, bne_ref, c_ref):
    c_ref[...] = (
        jnp.dot(ef_ref[...], w3t_ref[...], preferred_element_type=jnp.float32)
        + bne_ref[...])


def _tc_edgeproj(ef, w3t, bne):
    blk = 8000
    return pl.pallas_call(
        _tc_edgeproj_body,
        grid=(E // blk,),
        in_specs=[
            pl.BlockSpec((blk, DEF), lambda i: (i, 0)),
            pl.BlockSpec((DEF, DH), lambda i: (0, 0)),
            pl.BlockSpec((1, DH), lambda i: (0, 0)),
        ],
        out_specs=pl.BlockSpec((blk, DH), lambda i: (i, 0)),
        out_shape=jax.ShapeDtypeStruct((E, DH), jnp.float32),
    )(ef, w3t, bne)


def _tc_mid_body(p0_ref, p1_ref, cnt_ref, ones_ref, w1t_ref, w2t_ref,
                 a_ref, b_ref, psum_ref):
    i = pl.program_id(0)
    counts = lax.dot_general(cnt_ref[...], ones_ref[...],
                             (((0,), (0,)), ((), ())),
                             preferred_element_type=jnp.float32)
    cf = (p0_ref[...] + p1_ref[...]) / jnp.maximum(counts, 1.0)
    a_ref[...] = jnp.dot(cf, w1t_ref[...], preferred_element_type=jnp.float32)
    b_ref[...] = jnp.dot(cf, w2t_ref[...], preferred_element_type=jnp.float32)
    ps = jnp.sum(cf, axis=0, keepdims=True)

    @pl.when(i == 0)
    def _():
        psum_ref[...] = ps

    @pl.when(i > 0)
    def _():
        psum_ref[...] += ps


def _tc_mid(p0, p1, cnts, ones, w1t, w2t):
    blk = 2048
    return pl.pallas_call(
        _tc_mid_body,
        grid=(NPAD // blk,),
        in_specs=[
            pl.BlockSpec((blk, DH), lambda i: (i, 0)),
            pl.BlockSpec((blk, DH), lambda i: (i, 0)),
            pl.BlockSpec((NW, blk), lambda i: (0, i)),
            pl.BlockSpec((NW, 1), lambda i: (0, 0)),
            pl.BlockSpec((DH, DH), lambda i: (0, 0)),
            pl.BlockSpec((DH, DH), lambda i: (0, 0)),
        ],
        out_specs=[
            pl.BlockSpec((blk, DH), lambda i: (i, 0)),
            pl.BlockSpec((blk, DH), lambda i: (i, 0)),
            pl.BlockSpec((1, DH), lambda i: (0, 0)),
        ],
        out_shape=[
            jax.ShapeDtypeStruct((NPAD, DH), jnp.float32),
            jax.ShapeDtypeStruct((NPAD, DH), jnp.float32),
            jax.ShapeDtypeStruct((1, DH), jnp.float32),
        ],
    )(p0, p1, cnts, ones, w1t, w2t)


def _tc_head_body(p0_ref, p1_ref, p2_ref, esum_ref, wt0_ref, wt1_ref, wt2_ref,
                  bp_ref, out_ref):
    acc = (jnp.dot(p0_ref[...], wt0_ref[...], preferred_element_type=jnp.float32)
           + jnp.dot(p1_ref[...], wt1_ref[...], preferred_element_type=jnp.float32)
           + jnp.dot(p2_ref[...], wt2_ref[...], preferred_element_type=jnp.float32))
    out_ref[...] = jnp.maximum(acc / esum_ref[0, 0] + bp_ref[...], 0.0)


def _tc_head(p0, p1, p2, esum, wt0, wt1, wt2, bp):
    return pl.pallas_call(
        _tc_head_body,
        out_shape=jax.ShapeDtypeStruct((1, DF), jnp.float32),
    )(p0, p1, p2, esum, wt0, wt1, wt2, bp)


_sc_mesh = plsc.VectorSubcoreMesh(
    core_axis_name="c", subcore_axis_name="s", num_cores=NC, num_subcores=NS)


@functools.partial(
    pl.kernel,
    out_type=jax.ShapeDtypeStruct((NW * NPAD,), jnp.float32),
    mesh=_sc_mesh,
    compiler_params=pltpu.CompilerParams(needs_layout_passes=False),
    scratch_types=[
        pltpu.VMEM((KC,), jnp.int32),
        pltpu.VMEM((NPAD,), jnp.float32),
    ],
)
def _sc_count(src_hbm, outc_hbm, src_v, cnt_v):
    cid = lax.axis_index("c")
    sid = lax.axis_index("s")
    wid = cid * NS + sid

    zero16 = jnp.zeros((16,), jnp.float32)
    one16 = jnp.ones((16,), jnp.float32)

    def _zcnt(r, carry):
        cnt_v[pl.ds(r * 16, 16)] = zero16
        return carry

    lax.fori_loop(0, NPAD // 16, _zcnt, 0)

    ebase = wid * EPW

    def _block(j, carry):
        pltpu.sync_copy(src_hbm.at[pl.ds(ebase + j * KC, KC)], src_v)
        for q in range(KC // 16):
            plsc.addupdate_scatter(cnt_v, [src_v[pl.ds(q * 16, 16)]], one16)
        return carry

    lax.fori_loop(0, EPW // KC, _block, 0)
    pltpu.sync_copy(cnt_v, outc_hbm.at[pl.ds(wid * NPAD, NPAD)])


@functools.partial(
    pl.kernel,
    out_type=jax.ShapeDtypeStruct((NC * NPAD, DH), jnp.float32),
    mesh=_sc_mesh,
    compiler_params=pltpu.CompilerParams(needs_layout_passes=False),
    scratch_types=[
        pltpu.VMEM((K,), jnp.int32),
        pltpu.VMEM((K,), jnp.int32),
        pltpu.VMEM((K, DH), jnp.float32),
        pltpu.VMEM((K, DH), jnp.float32),
        pltpu.VMEM((K, DH), jnp.float32),
        pltpu.VMEM((K,), jnp.int32),
        pltpu.VMEM((K,), jnp.int32),
        pltpu.VMEM((K, DH), jnp.float32),
        pltpu.VMEM((K, DH), jnp.float32),
        pltpu.VMEM((K, DH), jnp.float32),
        pltpu.VMEM((K, DH), jnp.float32),
        pltpu.VMEM((K, DH), jnp.float32),
        pltpu.VMEM((K,), jnp.int32),
        pltpu.VMEM((K,), jnp.int32),
        pltpu.VMEM_SHARED((NPAD, DH), jnp.float32),
        pltpu.SemaphoreType.DMA,
        pltpu.SemaphoreType.DMA,
        pltpu.SemaphoreType.DMA,
        pltpu.SemaphoreType.DMA,
        pltpu.SemaphoreType.DMA,
        pltpu.SemaphoreType.DMA,
        pltpu.SemaphoreType.DMA,
        pltpu.SemaphoreType.DMA,
        pltpu.SemaphoreType.DMA,
        pltpu.SemaphoreType.DMA,
        pltpu.SemaphoreType.DMA,
        pltpu.SemaphoreType.DMA,
    ],
)
def _sc_edge(a_hbm, b_hbm, c_hbm, src_hbm, dst_hbm, out_hbm,
             src0, dst0, ra0, rb0, rc0, src1, dst1, ra1, rb1, rc1,
             ov0, ov1, sx0, sx1, acc_sh,
             sa0, sb0, sc0, sa1, sb1, sc1, si0, si1, sd0, sd1, ss0, ss1):
    cid = lax.axis_index("c")
    sid = lax.axis_index("s")
    wid = cid * NS + sid

    srcs = (src0, src1)
    dsts = (dst0, dst1)
    ras = (ra0, ra1)
    rbs = (rb0, rb1)
    rcs = (rc0, rc1)
    ovs = (ov0, ov1)
    sxs = (sx0, sx1)
    sss = (ss0, ss1)
    sas = (sa0, sa1)
    sbs = (sb0, sb1)
    scs = (sc0, sc1)
    sis = (si0, si1)
    sds = (sd0, sd1)

    zero16 = jnp.zeros((16,), jnp.float32)

    def _zrow(r, carry):
        for c in range(DH // 16):
            ov0[r, pl.ds(c * 16, 16)] = zero16
        return carry

    lax.fori_loop(0, K, _zrow, 0)
    for z in range(RPT // K):
        pltpu.sync_copy(ov0, acc_sh.at[pl.ds(sid * RPT + z * K, K)])
    plsc.subcore_barrier()

    ebase = wid * EPW

    def _issue_idx(j, b):
        eb = ebase + j * K
        pltpu.async_copy(src_hbm.at[pl.ds(eb, K)], srcs[b], sis[b])
        pltpu.async_copy(dst_hbm.at[pl.ds(eb, K)], dsts[b], sds[b])

    def _wait_idx(b):
        pltpu.make_async_copy(src_hbm.at[pl.ds(0, K)], srcs[b], sis[b]).wait()
        pltpu.make_async_copy(dst_hbm.at[pl.ds(0, K)], dsts[b], sds[b]).wait()

    def _issue_rows(j, b):
        eb = ebase + j * K
        pltpu.async_copy(a_hbm.at[srcs[b]], ras[b], sas[b])
        pltpu.async_copy(b_hbm.at[dsts[b]], rbs[b], sbs[b])
        pltpu.async_copy(c_hbm.at[pl.ds(eb, K)], rcs[b], scs[b])

    def _copy_sidx(b):
        sxs[b][pl.ds(0, 16)] = srcs[b][pl.ds(0, 16)]
        sxs[b][pl.ds(16, 16)] = srcs[b][pl.ds(16, 16)]
        sxs[b][pl.ds(24, 16)] = srcs[b][pl.ds(24, 16)]

    for b in range(2):
        _issue_idx(b, b)
        _wait_idx(b)
        _copy_sidx(b)
        _issue_rows(b, b)

    def _round(g, carry):
        for b in range(2):
            j = 2 * g + b
            pltpu.make_async_copy(a_hbm.at[srcs[b]], ras[b], sas[b]).wait()
            pltpu.make_async_copy(b_hbm.at[dsts[b]], rbs[b], sbs[b]).wait()
            pltpu.make_async_copy(c_hbm.at[pl.ds(0, K)], rcs[b],
                                  scs[b]).wait()

            @pl.when(j + 2 < NBLK)
            def _():
                _issue_idx(j + 2, b)

            @pl.when(g >= 1)
            def _():
                pltpu.make_async_copy(ovs[b], acc_sh.at[sxs[b]],
                                      sss[b]).wait()

            ra, rb, rc, ov = ras[b], rbs[b], rcs[b], ovs[b]

            def _row(r, rc_):
                for c in range(DH // 16):
                    s = pl.ds(c * 16, 16)
                    ov[r, s] = jnp.maximum(
                        ra[r, s] + rb[r, s] + rc[r, s], 0.0)
                return rc_

            lax.fori_loop(0, K, _row, 0)
            pltpu.async_copy(ov, acc_sh.at[sxs[b]], sss[b], add=True)

            @pl.when(j + 2 < NBLK)
            def _():
                _wait_idx(b)
                _copy_sidx(b)
                _issue_rows(j + 2, b)
        return carry

    lax.fori_loop(0, NBLK // 2, _round, 0)
    for b in range(2):
        pltpu.make_async_copy(ovs[b], acc_sh.at[sxs[b]], sss[b]).wait()
    plsc.subcore_barrier()

    for z in range(RPT // K):
        rs = sid * RPT + z * K
        pltpu.sync_copy(acc_sh.at[pl.ds(rs, K)], ov0)
        pltpu.sync_copy(ov0, out_hbm.at[pl.ds(cid * NPAD + rs, K)])


def kernel(child_feats, child_exists, edge_type_onehot, edge_feats,
           edge_indices, W_child, b_child, W_ne, b_ne, W_parent, b_parent):
    child = child_feats[0]
    exists = child_exists[0]
    ef = jnp.concatenate([edge_type_onehot[0], edge_feats[0]], axis=1)
    src = edge_indices[0, :, 0]
    dst = edge_indices[0, :, 1]

    wct = W_child.T
    w1t = W_ne[:, 0:DH].T
    w2t = W_ne[:, DH:2 * DH].T
    w3t = W_ne[:, 2 * DH:2 * DH + DEF].T
    bc = b_child.reshape(1, DH)
    bne = b_ne.reshape(1, DH)
    wt0 = W_parent[:, 0:DH].T
    wt1 = W_parent[:, DH:2 * DH].T
    wt2 = W_parent[:, 2 * DH:3 * DH].T
    bp = b_parent.reshape(1, DF)

    a0, b0, psum0, esum = _tc_pre(child, exists, wct, bc, w1t, w2t)
    c = _tc_edgeproj(ef, w3t, bne)

    ones_nw = jnp.ones((NW, 1), jnp.float32)
    cnts = _sc_count(src).reshape(NW, NPAD)

    sums1 = _sc_edge(a0, b0, c, src, dst)
    a1, b1, psum1 = _tc_mid(sums1[0:NPAD], sums1[NPAD:2 * NPAD],
                            cnts, ones_nw, w1t, w2t)

    sums2 = _sc_edge(a1, b1, c, src, dst)
    _, _, psum2 = _tc_mid(sums2[0:NPAD], sums2[NPAD:2 * NPAD],
                          cnts, ones_nw, w1t, w2t)

    return _tc_head(psum0, psum1, psum2, esum, wt0, wt1, wt2, bp)
